# Initial kernel scaffold; baseline (speedup 1.0000x reference)
import jax
import jax.numpy as jnp
from jax.experimental import pallas as pl


def kernel(points, features):
    raise NotImplementedError("write your pallas kernel here")


# speedup vs baseline: 3.4321x; 3.4321x over previous
import functools

import jax
import jax.numpy as jnp
from jax import lax
from jax.experimental import pallas as pl

_S = 64
_K = 8
_RADIUS = 2.0
_R = 2.0 * _RADIUS / float(_S)
_R2 = _R * _R
_B = 2
_N = 8192
_F = 64
_P = _S * _S
_TP = 128
_NT = _P // _TP


def _raster_body(pts_ref, feats_ref, fout_ref, zw_ref, vray_ref, idx_ref,
                 zbuf_ref, dist_ref, w_ref):
    b = pl.program_id(0)
    t = pl.program_id(1)

    pix_lin = t * _TP + lax.broadcasted_iota(jnp.int32, (_TP, 1), 0)
    row = pix_lin // _S
    col = pix_lin % _S
    pixx = 1.0 - (2.0 * col.astype(jnp.float32) + 1.0) / float(_S)
    pixy = 1.0 - (2.0 * row.astype(jnp.float32) + 1.0) / float(_S)

    pts = pts_ref[0]
    ptx = pts[0:1, :]
    pty = pts[1:2, :]
    ptz = pts[2:3, :]

    a = pixx * pixx + pixy * pixy
    bb = ptx * ptx + pty * pty
    bf = lambda v: v.astype(jnp.bfloat16).astype(jnp.float32)
    c = bf(pixx) * bf(ptx) + bf(pixy) * bf(pty)
    d2 = (a + bb) - 2.0 * c

    valid = (d2 <= _R2) & (ptz > 0.0)
    zbig = jnp.where(valid, jnp.broadcast_to(ptz, d2.shape), jnp.inf)
    iota_n = lax.broadcasted_iota(jnp.int32, (_TP, _N), 1)

    wmat = jnp.zeros((_TP, _N), jnp.float32)
    idx_cols = []
    zbuf_cols = []
    dist_cols = []
    w_cols = []
    found0 = None
    for _ in range(_K):
        zmin = jnp.min(zbig, axis=1, keepdims=True)
        found = zmin < jnp.inf
        amin = jnp.min(jnp.where(zbig == zmin, iota_n, _N), axis=1,
                       keepdims=True)
        onehot = iota_n == amin
        d2sel = jnp.max(jnp.where(onehot, d2, -1.0), axis=1,
                        keepdims=True)
        zbig = jnp.where(onehot, jnp.inf, zbig)
        wsel = jnp.clip(jnp.exp(-jnp.maximum(d2sel / _R2, 0.0)), 0.0, 0.99)
        wmat = jnp.where(onehot & found, wsel, wmat)
        idx_cols.append(jnp.where(found, amin + b * _N, -1))
        zbuf_cols.append(jnp.where(found, zmin, -1.0))
        dist_cols.append(jnp.where(found, d2sel, -1.0))
        w_cols.append(jnp.where(found, wsel, 0.99))
        if found0 is None:
            found0 = found

    zbuf_tk = jnp.concatenate(zbuf_cols, axis=1)
    dist_tk = jnp.concatenate(dist_cols, axis=1)
    idx_tk = jnp.concatenate(idx_cols, axis=1)
    w_tk = jnp.concatenate(w_cols, axis=1)

    fout = lax.dot_general(feats_ref[0], wmat,
                           dimension_numbers=(((0,), (1,)), ((), ())),
                           precision=lax.Precision.HIGHEST,
                           preferred_element_type=jnp.float32)
    fout_ref[0] = fout

    wn = jnp.where(idx_tk >= 0, w_tk, 0.0)
    denom = jnp.maximum(jnp.sum(wn, axis=1, keepdims=True), 1e-9)
    wn = wn / denom
    zw = jnp.sum(zbuf_tk * wn, axis=1, keepdims=True)

    zw_ref[0] = zw.reshape(1, _TP)
    vray_ref[0] = found0.astype(jnp.float32).reshape(1, _TP)
    idx_ref[0] = idx_tk.T
    zbuf_ref[0] = zbuf_tk
    dist_ref[0] = dist_tk
    w_ref[0] = w_tk.T


@jax.jit
def kernel(points, features):
    pts_t = jnp.transpose(points, (0, 2, 1))

    grid = (_B, _NT)
    out_shapes = (
        jax.ShapeDtypeStruct((_B, _F, _P), jnp.float32),
        jax.ShapeDtypeStruct((_B, 1, _P), jnp.float32),
        jax.ShapeDtypeStruct((_B, 1, _P), jnp.float32),
        jax.ShapeDtypeStruct((_B, _K, _P), jnp.int32),
        jax.ShapeDtypeStruct((_B, _P, _K), jnp.float32),
        jax.ShapeDtypeStruct((_B, _P, _K), jnp.float32),
        jax.ShapeDtypeStruct((_B, _K, _P), jnp.float32),
    )
    in_specs = [
        pl.BlockSpec((1, 3, _N), lambda b, t: (b, 0, 0)),
        pl.BlockSpec((1, _N, _F), lambda b, t: (b, 0, 0)),
    ]
    out_specs = (
        pl.BlockSpec((1, _F, _TP), lambda b, t: (b, 0, t)),
        pl.BlockSpec((1, 1, _TP), lambda b, t: (b, 0, t)),
        pl.BlockSpec((1, 1, _TP), lambda b, t: (b, 0, t)),
        pl.BlockSpec((1, _K, _TP), lambda b, t: (b, 0, t)),
        pl.BlockSpec((1, _TP, _K), lambda b, t: (b, t, 0)),
        pl.BlockSpec((1, _TP, _K), lambda b, t: (b, t, 0)),
        pl.BlockSpec((1, _K, _TP), lambda b, t: (b, 0, t)),
    )
    fout, zw, vray, idx, zbuf, dist, w = pl.pallas_call(
        _raster_body,
        grid=grid,
        in_specs=in_specs,
        out_specs=out_specs,
        out_shape=out_shapes,
    )(pts_t, features)

    feats_out = fout.reshape(_B, _F, _S, _S)
    z_weighted = zw.reshape(_B, 1, _S, _S)
    valid_ray = vray.reshape(_B, _S, _S)
    idx_o = idx.reshape(_B, _K, _S, _S)
    zbuf_o = zbuf.reshape(_B, _S, _S, _K)
    dist_o = dist.reshape(_B, _S, _S, _K)
    w_o = w.reshape(_B, _K, _S, _S)
    mean_ray = jnp.mean(valid_ray, axis=(1, 2))
    mean_pts = jnp.mean((idx_o >= 0).astype(jnp.float32), axis=(1, 2, 3))
    return (feats_out, z_weighted, valid_ray, mean_ray, mean_pts,
            idx_o, zbuf_o, dist_o, w_o)
